# baseline (speedup 1.0000x reference)
import functools

import jax
import jax.numpy as jnp
from jax import lax
from jax.experimental import pallas as pl
from jax.experimental.pallas import tpu as pltpu
from jax.experimental.pallas import tpu_sc as plsc

_VOCAB = 1000000
_EMB = 64
_SEQ = 200
_BATCH = 1024
_ROWS = _SEQ * _BATCH
_NW = 32
_CHUNK = 128
_NCHUNK = _ROWS // _CHUNK
_CPW = _NCHUNK // _NW
_CPP = _BATCH // _CHUNK
_LANES = 16
_VPR = _EMB // _LANES


def _pos_enc() -> jax.Array:
    i = jnp.arange(_SEQ, dtype=jnp.float32)[:, None]
    j = jnp.arange(_EMB)
    even = (j % 2 == 0)
    exponent = jnp.where(even, j, j - 1).astype(jnp.float32) / float(_EMB)
    angle = i / (10000.0 ** exponent)
    return jnp.where(even[None, :], jnp.sin(angle), jnp.cos(angle))


_MESH = plsc.VectorSubcoreMesh(core_axis_name="c", subcore_axis_name="s")


@functools.partial(
    pl.kernel,
    mesh=_MESH,
    out_type=jax.ShapeDtypeStruct((_ROWS, _EMB), jnp.float32),
    scratch_types=[
        pltpu.VMEM((_CHUNK,), jnp.int32),
        pltpu.VMEM((_CHUNK, _EMB), jnp.float32),
        pltpu.VMEM((_SEQ, _EMB), jnp.float32),
        pltpu.SemaphoreType.DMA,
    ],
    compiler_params=pltpu.CompilerParams(use_tc_tiling_on_sc=False),
)
def _emb_kernel(x_hbm, table_hbm, pe_hbm, out_hbm, idx_v, rows_v, pe_v, sem):
    wid = lax.axis_index("s") * 2 + lax.axis_index("c")
    pltpu.sync_copy(pe_hbm, pe_v)

    def chunk_body(k, carry):
        c = wid * _CPW + k
        base = c * _CHUNK
        s = c // _CPP
        pltpu.sync_copy(x_hbm.at[pl.ds(base, _CHUNK)], idx_v)
        pltpu.async_copy(table_hbm.at[idx_v], rows_v, sem).wait()
        pe_regs = [pe_v[s, pl.ds(j * _LANES, _LANES)] for j in range(_VPR)]

        def row_body(r, rcarry):
            for j in range(_VPR):
                sl = pl.ds(j * _LANES, _LANES)
                rows_v[r, sl] = rows_v[r, sl] + pe_regs[j]
            return rcarry

        lax.fori_loop(0, _CHUNK, row_body, 0, unroll=4)
        pltpu.sync_copy(rows_v, out_hbm.at[pl.ds(base, _CHUNK)])
        return carry

    lax.fori_loop(0, _CPW, chunk_body, 0)


def kernel(x, table):
    pe = _pos_enc()
    out = _emb_kernel(x.reshape(_ROWS).astype(jnp.int32), table, pe)
    return out.reshape(_SEQ, _BATCH, _EMB)


# speedup vs baseline: 1.0927x; 1.0927x over previous
import functools

import jax
import jax.numpy as jnp
from jax import lax
from jax.experimental import pallas as pl
from jax.experimental.pallas import tpu as pltpu
from jax.experimental.pallas import tpu_sc as plsc

_VOCAB = 1000000
_EMB = 64
_SEQ = 200
_BATCH = 1024
_ROWS = _SEQ * _BATCH
_NW = 32
_CHUNK = 128
_NCHUNK = _ROWS // _CHUNK
_CPW = _NCHUNK // _NW
_CPP = _BATCH // _CHUNK
_LANES = 16
_VPR = _EMB // _LANES
_NBUF = 5
_KK = _CPW // _NBUF


def _pos_enc() -> jax.Array:
    i = jnp.arange(_SEQ, dtype=jnp.float32)[:, None]
    j = jnp.arange(_EMB)
    even = (j % 2 == 0)
    exponent = jnp.where(even, j, j - 1).astype(jnp.float32) / float(_EMB)
    angle = i / (10000.0 ** exponent)
    return jnp.where(even[None, :], jnp.sin(angle), jnp.cos(angle))


_MESH = plsc.VectorSubcoreMesh(core_axis_name="c", subcore_axis_name="s")


@functools.partial(
    pl.kernel,
    mesh=_MESH,
    out_type=jax.ShapeDtypeStruct((_ROWS, _EMB), jnp.float32),
    scratch_types=[
        pltpu.VMEM((_CPW, _CHUNK), jnp.int32),
        pltpu.VMEM((_NBUF, _CHUNK, _EMB), jnp.float32),
        pltpu.VMEM((_SEQ, _EMB), jnp.float32),
    ]
    + [pltpu.SemaphoreType.DMA] * (2 * _NBUF),
    compiler_params=pltpu.CompilerParams(use_tc_tiling_on_sc=False),
)
def _emb_kernel(x_hbm, table_hbm, pe_hbm, out_hbm, idx_v, rows_v, pe_v, *sems):
    gsem = sems[:_NBUF]
    osem = sems[_NBUF:]
    wid = lax.axis_index("s") * 2 + lax.axis_index("c")
    c0 = wid * _CPW
    pltpu.sync_copy(pe_hbm, pe_v)
    pltpu.sync_copy(x_hbm.at[wid], idx_v)

    def _gather(k, b):
        return pltpu.async_copy(
            table_hbm.at[idx_v.at[k]], rows_v.at[b], gsem[b])

    def _gather_wait(k, b):
        pltpu.make_async_copy(
            table_hbm.at[idx_v.at[k]], rows_v.at[b], gsem[b]).wait()

    def _out(k, b):
        return pltpu.async_copy(
            rows_v.at[b], out_hbm.at[pl.ds((c0 + k) * _CHUNK, _CHUNK)],
            osem[b])

    def _out_wait(k, b):
        pltpu.make_async_copy(
            rows_v.at[b], out_hbm.at[pl.ds((c0 + k) * _CHUNK, _CHUNK)],
            osem[b]).wait()

    _gather(0, 0)
    _gather(1, 1)

    def outer(kk, carry):
        for b in range(_NBUF):
            k = kk * _NBUF + b
            _gather_wait(k, b)
            s = (c0 + k) // _CPP
            pe_regs = [pe_v[s, pl.ds(j * _LANES, _LANES)]
                       for j in range(_VPR)]

            @plsc.parallel_loop(0, _CHUNK, 1, unroll=4)
            def add_body(r):
                for j in range(_VPR):
                    sl = pl.ds(j * _LANES, _LANES)
                    rows_v[b, r, sl] = rows_v[b, r, sl] + pe_regs[j]

            _out(k, b)
            b2 = (b + 2) % _NBUF
            if b < 3:
                @pl.when(kk >= 1)
                def _():
                    _out_wait(k - 3, b2)
            else:
                _out_wait(k - 3, b2)
            if b < 3:
                _gather(k + 2, b2)
            else:
                @pl.when(kk <= _KK - 2)
                def _():
                    _gather(k + 2, b2)
        return carry

    lax.fori_loop(0, _KK, outer, 0)
    for b, k in ((2, _CPW - 3), (3, _CPW - 2), (4, _CPW - 1)):
        _out_wait(k, b)


def kernel(x, table):
    pe = _pos_enc()
    out = _emb_kernel(
        x.reshape(_NW, _CPW, _CHUNK).astype(jnp.int32), table, pe)
    return out.reshape(_SEQ, _BATCH, _EMB)
